# baseline (speedup 1.0000x reference)
import functools

import jax
import jax.numpy as jnp
from jax import lax
from jax.experimental import pallas as pl
from jax.experimental.pallas import tpu as pltpu
from jax.experimental.pallas import tpu_sc as plsc

_N = 8192
_D = 256
_BETA = 0.25
_ROWS = 256
_COLS = 2048
_NJ = _N // _COLS

_NC, _NS, _L = 2, 16, 16
_NW = _NC * _NS
_BPW = _N // _NW
_CH = 128
_NCH = _BPW // _CH
_ZCH = _N // _NS


def _vq_body(zf_ref, zn_ref, zsq_ref, cn_ref, csq_ref, prob_ref, idx_ref,
             mind_ref):
    zn = zf_ref[...] / zn_ref[...]
    zsq = zsq_ref[...]
    rmin = jnp.full((_ROWS, 1), jnp.inf, jnp.float32)
    rarg = jnp.zeros((_ROWS, 1), jnp.int32)
    col = lax.broadcasted_iota(jnp.int32, (_ROWS, _COLS), 1)
    acc = jnp.zeros((_ROWS, 1), jnp.float32)
    for jj in range(_NJ):
        cn = cn_ref[jj * _COLS:(jj + 1) * _COLS, :]
        s = lax.dot_general(zn, cn, (((1,), (1,)), ((), ())),
                            preferred_element_type=jnp.float32)
        csq = csq_ref[0:1, jj * _COLS:(jj + 1) * _COLS]
        dist = zsq + csq - 2.0 * s
        lm = jnp.min(dist, axis=1, keepdims=True)
        lidx = (jnp.min(jnp.where(dist == lm, col, _COLS), axis=1, keepdims=True)
                + jj * _COLS)
        e = jnp.exp(-dist)
        prob_ref[:, jj * _COLS:(jj + 1) * _COLS] = e
        acc = acc + jnp.sum(e, axis=1, keepdims=True)
        better = lm < rmin
        rmin = jnp.where(better, lm, rmin)
        rarg = jnp.where(better, lidx, rarg)
    inv = 1.0 / acc
    for jj in range(_NJ):
        prob_ref[:, jj * _COLS:(jj + 1) * _COLS] = (
            prob_ref[:, jj * _COLS:(jj + 1) * _COLS] * inv)
    idx_ref[...] = rarg
    mind_ref[...] = rmin


_vq_call = pl.pallas_call(
    _vq_body,
    grid=(_N // _ROWS,),
    in_specs=[
        pl.BlockSpec((_ROWS, _D), lambda i: (i, 0)),
        pl.BlockSpec((_ROWS, 1), lambda i: (i, 0)),
        pl.BlockSpec((_ROWS, 1), lambda i: (i, 0)),
        pl.BlockSpec((_N, _D), lambda i: (0, 0)),
        pl.BlockSpec((1, _N), lambda i: (0, 0)),
    ],
    out_specs=[
        pl.BlockSpec((_ROWS, _N), lambda i: (i, 0)),
        pl.BlockSpec((_ROWS, 1), lambda i: (i, 0)),
        pl.BlockSpec((_ROWS, 1), lambda i: (i, 0)),
    ],
    out_shape=[
        jax.ShapeDtypeStruct((_N, _N), jnp.float32),
        jax.ShapeDtypeStruct((_N, 1), jnp.int32),
        jax.ShapeDtypeStruct((_N, 1), jnp.float32),
    ],
)


def _sc_body(cn_hbm, idx_hbm, q_hbm, cnt_hbm,
             idx_v, rows_v, ones_v, zeros_v, cnt_sh, sem0, sem1):
    cid = lax.axis_index("c")
    sid = lax.axis_index("s")
    wid = sid * _NC + cid
    base = wid * _BPW
    pltpu.sync_copy(idx_hbm.at[wid], idx_v)
    cp0 = pltpu.async_copy(cn_hbm.at[idx_v.at[0]], rows_v.at[pl.ds(0, _CH)], sem0)
    cp1 = pltpu.async_copy(cn_hbm.at[idx_v.at[1]], rows_v.at[pl.ds(_CH, _CH)], sem1)
    for t in range(_CH // _L):
        ones_v[pl.ds(t * _L, _L)] = jnp.full((_L,), 1, jnp.int32)
    for t in range(_ZCH // _L):
        zeros_v[pl.ds(t * _L, _L)] = jnp.full((_L,), 0, jnp.int32)
    pltpu.sync_copy(zeros_v, cnt_sh.at[pl.ds(sid * _ZCH, _ZCH)])
    plsc.subcore_barrier()
    pltpu.sync_copy(ones_v, cnt_sh.at[idx_v.at[0]], add=True)
    pltpu.sync_copy(ones_v, cnt_sh.at[idx_v.at[1]], add=True)
    plsc.subcore_barrier()

    @pl.when(sid == 0)
    def _():
        pltpu.sync_copy(cnt_sh, cnt_hbm.at[cid])

    cp0.wait()
    cp1.wait()
    pltpu.sync_copy(rows_v, q_hbm.at[pl.ds(base, _BPW)])


@functools.cache
def _sc_gather_count():
    return pl.kernel(
        _sc_body,
        mesh=plsc.VectorSubcoreMesh(core_axis_name="c", subcore_axis_name="s",
                                    num_cores=_NC, num_subcores=_NS),
        out_type=[
            jax.ShapeDtypeStruct((_N, _D), jnp.float32),
            jax.ShapeDtypeStruct((_NC, _N), jnp.int32),
        ],
        scratch_types=[
            pltpu.VMEM((_NCH, _CH), jnp.int32),
            pltpu.VMEM((_BPW, _D), jnp.float32),
            pltpu.VMEM((_CH,), jnp.int32),
            pltpu.VMEM((_ZCH,), jnp.int32),
            pltpu.VMEM_SHARED((_N,), jnp.int32),
            pltpu.SemaphoreType.DMA,
            pltpu.SemaphoreType.DMA,
        ],
    )


def _l2_normalize_rows(x):
    n = jnp.sqrt(jnp.sum(x * x, axis=1, keepdims=True))
    return x / jnp.maximum(n, 1e-12)


def kernel(z, codebook):
    z_perm = jnp.transpose(z, (0, 2, 3, 1))
    b, h, w, d = z_perm.shape
    z_flat = z_perm.reshape(-1, d)

    zden = jnp.maximum(
        jnp.sqrt(jnp.sum(z_flat * z_flat, axis=1, keepdims=True)), 1e-12)
    zsq = jnp.sum((z_flat / zden) ** 2, axis=1, keepdims=True)
    c_norm = _l2_normalize_rows(codebook)
    csq = jnp.sum(c_norm ** 2, axis=1)[None, :]

    distance_prob, idx2, mind2 = _vq_call(z_flat, zden, zsq, c_norm, csq)

    idx3 = idx2.reshape(_NW, _NCH, _CH)
    q_flat, cnt2 = _sc_gather_count()(c_norm, idx3)
    vq_current_count = cnt2[0] + cnt2[1]

    mse = jnp.sum(mind2) / (_N * _D)
    codebook_loss = mse
    commitment_loss = mse
    loss = codebook_loss + _BETA * commitment_loss

    q = jnp.transpose(q_flat.reshape(b, h, w, d), (0, 3, 1, 2))

    return (q, loss, codebook_loss, commitment_loss, distance_prob,
            vq_current_count)


# speedup vs baseline: 1.0320x; 1.0320x over previous
import functools

import jax
import jax.numpy as jnp
from jax import lax
from jax.experimental import pallas as pl
from jax.experimental.pallas import tpu as pltpu
from jax.experimental.pallas import tpu_sc as plsc

_N = 8192
_D = 256
_BETA = 0.25
_ROWS = 256
_COLS = 2048
_NJ = _N // _COLS

_NC, _NS, _L = 2, 16, 16
_NW = _NC * _NS
_BPW = _N // _NW
_CH = 128
_NCH = _BPW // _CH
_ZCH = _N // _NS


def _vq_body(zn_ref, zsq_ref, cn_ref, csq_ref, prob_ref, idx_ref, mind_ref):
    zn = zn_ref[...]
    zsq = zsq_ref[...]
    rmin = jnp.full((_ROWS, 1), jnp.inf, jnp.float32)
    rarg = jnp.zeros((_ROWS, 1), jnp.int32)
    col = lax.broadcasted_iota(jnp.int32, (_ROWS, _COLS), 1)
    lms = []
    sums = []
    for jj in range(_NJ):
        cn = cn_ref[jj * _COLS:(jj + 1) * _COLS, :]
        s = lax.dot_general(zn, cn, (((1,), (1,)), ((), ())),
                            preferred_element_type=jnp.float32)
        csq = csq_ref[0:1, jj * _COLS:(jj + 1) * _COLS]
        dist = zsq + csq - 2.0 * s
        lm = jnp.min(dist, axis=1, keepdims=True)
        lidx = (jnp.min(jnp.where(dist == lm, col, _COLS), axis=1, keepdims=True)
                + jj * _COLS)
        e = jnp.exp(lm - dist)
        prob_ref[:, jj * _COLS:(jj + 1) * _COLS] = e
        lms.append(lm)
        sums.append(jnp.sum(e, axis=1, keepdims=True))
        better = lm < rmin
        rmin = jnp.where(better, lm, rmin)
        rarg = jnp.where(better, lidx, rarg)
    acc = jnp.zeros((_ROWS, 1), jnp.float32)
    for jj in range(_NJ):
        acc = acc + sums[jj] * jnp.exp(rmin - lms[jj])
    inv = 1.0 / acc
    for jj in range(_NJ):
        scale = jnp.exp(rmin - lms[jj]) * inv
        prob_ref[:, jj * _COLS:(jj + 1) * _COLS] = (
            prob_ref[:, jj * _COLS:(jj + 1) * _COLS] * scale)
    idx_ref[...] = rarg
    mind_ref[...] = rmin


_vq_call = pl.pallas_call(
    _vq_body,
    grid=(_N // _ROWS,),
    in_specs=[
        pl.BlockSpec((_ROWS, _D), lambda i: (i, 0)),
        pl.BlockSpec((_ROWS, 1), lambda i: (i, 0)),
        pl.BlockSpec((_N, _D), lambda i: (0, 0)),
        pl.BlockSpec((1, _N), lambda i: (0, 0)),
    ],
    out_specs=[
        pl.BlockSpec((_ROWS, _N), lambda i: (i, 0)),
        pl.BlockSpec((_ROWS, 1), lambda i: (i, 0)),
        pl.BlockSpec((_ROWS, 1), lambda i: (i, 0)),
    ],
    out_shape=[
        jax.ShapeDtypeStruct((_N, _N), jnp.float32),
        jax.ShapeDtypeStruct((_N, 1), jnp.int32),
        jax.ShapeDtypeStruct((_N, 1), jnp.float32),
    ],
)


def _sc_body(cn_hbm, idx_hbm, q_hbm, cnt_hbm,
             idx_v, idxc_v, rows_v, ones_v, zeros_v, cnt_sh, sem0, sem1):
    cid = lax.axis_index("c")
    sid = lax.axis_index("s")
    wid = sid * _NC + cid
    base = wid * _BPW
    pltpu.sync_copy(idx_hbm.at[wid], idx_v)
    cp0 = pltpu.async_copy(cn_hbm.at[idx_v.at[0]], rows_v.at[pl.ds(0, _CH)], sem0)
    cp1 = pltpu.async_copy(cn_hbm.at[idx_v.at[1]], rows_v.at[pl.ds(_CH, _CH)], sem1)
    for t in range(_CH // _L):
        ones_v[pl.ds(t * _L, _L)] = jnp.full((_L,), 1, jnp.int32)
    for t in range(_ZCH // _L):
        zeros_v[pl.ds(t * _L, _L)] = jnp.full((_L,), 0, jnp.int32)

    @pl.when(cid == 0)
    def _():
        pltpu.sync_copy(zeros_v, cnt_sh.at[pl.ds(sid * _ZCH, _ZCH)])
        pltpu.sync_copy(idx_hbm.at[2 * sid], idxc_v.at[pl.ds(0, _NCH)])
        pltpu.sync_copy(idx_hbm.at[2 * sid + 1], idxc_v.at[pl.ds(_NCH, _NCH)])

    plsc.subcore_barrier()

    @pl.when(cid == 0)
    def _():
        for t in range(2 * _NCH):
            pltpu.sync_copy(ones_v, cnt_sh.at[idxc_v.at[t]], add=True)

    plsc.subcore_barrier()

    @pl.when(jnp.logical_and(cid == 0, sid == 0))
    def _():
        pltpu.sync_copy(cnt_sh, cnt_hbm)

    cp0.wait()
    cp1.wait()
    pltpu.sync_copy(rows_v, q_hbm.at[pl.ds(base, _BPW)])


@functools.cache
def _sc_gather_count():
    return pl.kernel(
        _sc_body,
        mesh=plsc.VectorSubcoreMesh(core_axis_name="c", subcore_axis_name="s",
                                    num_cores=_NC, num_subcores=_NS),
        out_type=[
            jax.ShapeDtypeStruct((_N, _D), jnp.float32),
            jax.ShapeDtypeStruct((_N,), jnp.int32),
        ],
        scratch_types=[
            pltpu.VMEM((_NCH, _CH), jnp.int32),
            pltpu.VMEM((2 * _NCH, _CH), jnp.int32),
            pltpu.VMEM((_BPW, _D), jnp.float32),
            pltpu.VMEM((_CH,), jnp.int32),
            pltpu.VMEM((_ZCH,), jnp.int32),
            pltpu.VMEM_SHARED((_N,), jnp.int32),
            pltpu.SemaphoreType.DMA,
            pltpu.SemaphoreType.DMA,
        ],
    )


def _l2_normalize_rows(x):
    n = jnp.sqrt(jnp.sum(x * x, axis=1, keepdims=True))
    return x / jnp.maximum(n, 1e-12)


def kernel(z, codebook):
    z_perm = jnp.transpose(z, (0, 2, 3, 1))
    b, h, w, d = z_perm.shape
    z_flat = z_perm.reshape(-1, d)

    z_norm = _l2_normalize_rows(z_flat)
    c_norm = _l2_normalize_rows(codebook)
    zsq = jnp.sum(z_norm ** 2, axis=1, keepdims=True)
    csq = jnp.sum(c_norm ** 2, axis=1)[None, :]

    distance_prob, idx2, mind2 = _vq_call(z_norm, zsq, c_norm, csq)

    idx3 = idx2.reshape(_NW, _NCH, _CH)
    q_flat, vq_current_count = _sc_gather_count()(c_norm, idx3)

    mse = jnp.sum(mind2) / (_N * _D)
    codebook_loss = mse
    commitment_loss = mse
    loss = codebook_loss + _BETA * commitment_loss

    q = jnp.transpose(q_flat.reshape(b, h, w, d), (0, 3, 1, 2))

    return (q, loss, codebook_loss, commitment_loss, distance_prob,
            vq_current_count)
